# baseline (speedup 1.0000x reference)
import functools

import jax
import jax.numpy as jnp
from jax import lax
from jax.experimental import pallas as pl
from jax.experimental.pallas import tpu as pltpu
from jax.experimental.pallas import tpu_sc as plsc

_NC = 2
_NS = 16
_NW = _NC * _NS

_GROUP = 128
_TCC = 1024


@functools.cache
def _build_tconv(V, D):
    def body(t_ref, o_ref):
        o_ref[...] = t_ref[...].T.reshape(_TCC * D // 128, 128)

    return pl.pallas_call(
        body,
        grid=(pl.cdiv(V, _TCC),),
        in_specs=[pl.BlockSpec((D, _TCC), lambda g: (0, g))],
        out_specs=pl.BlockSpec((_TCC * D // 128, 128), lambda g: (g, 0)),
        out_shape=jax.ShapeDtypeStruct((V * D // 128, 128), jnp.float32),
    )


@functools.cache
def _build_tpose(V, D):
    _CW = 768
    n_full = V // _CW
    tail = V - n_full * _CW
    n_iter = (n_full + _NW - 1) // _NW
    mesh = plsc.VectorSubcoreMesh(core_axis_name="c", subcore_axis_name="s")

    @functools.partial(
        pl.kernel,
        mesh=mesh,
        out_type=jax.ShapeDtypeStruct((V * D // 128, 128), jnp.float32),
        compiler_params=pltpu.CompilerParams(
            use_tc_tiling_on_sc=True, needs_layout_passes=False),
        scratch_types=[
            pltpu.VMEM((D, _CW), jnp.float32),
            pltpu.VMEM((D, _CW), jnp.float32),
            pltpu.VMEM((_CW * D // 128, 128), jnp.float32),
            pltpu.VMEM((_CW * D // 128, 128), jnp.float32),
            pltpu.VMEM((D, 64), jnp.float32),
            pltpu.VMEM((16, 128), jnp.float32),
            pltpu.SemaphoreType.DMA,
            pltpu.SemaphoreType.DMA,
            pltpu.SemaphoreType.DMA,
        ],
    )
    def ka(tt_hbm, out_hbm, in0_v, in1_v, ob0_v, ob1_v, int_v, outt_v,
           isem0, isem1, osem):
        wid = lax.axis_index("s") * _NC + lax.axis_index("c")
        lane = lax.iota(jnp.int32, 16)
        dvec = [lane, lane + 16]
        in_bufs = [in0_v, in1_v]
        out_bufs = [ob0_v, ob1_v]
        in_sems = [isem0, isem1]
        sup = _CW * D // 128

        mconsts = [(lane + k) & 15 for k in range(16)]

        def transpose_block(src, dst, ncol):
            for k in range(16):
                m = mconsts[k]
                mrow = lax.shift_right_logical(m, 2)
                mcols = [lax.shift_left(m & 3, 5) + lane + d0 * 16
                         for d0 in range(D // 16)]

                def cb_body(cb, carry, m=m, mrow=mrow, mcols=mcols):
                    lcol = m + cb * 16
                    srow = mrow + cb * 4
                    for d0 in range(D // 16):
                        x = plsc.load_gather(src, [dvec[d0], lcol])
                        plsc.store_scatter(dst, [srow, mcols[d0]], x)
                    return carry

                lax.fori_loop(0, ncol // 16, cb_body, 0)

        def fire(c, p):
            pltpu.async_copy(
                tt_hbm.at[:, pl.ds(c * _CW, _CW)], in_bufs[p], in_sems[p])

        def drain_in(p):
            pltpu.make_async_copy(
                tt_hbm.at[:, pl.ds(0, _CW)], in_bufs[p], in_sems[p]).wait()

        def drain_out(p):
            pltpu.make_async_copy(
                out_hbm.at[pl.ds(0, sup)], out_bufs[p], osem).wait()

        @pl.when(wid < n_full)
        def _():
            fire(wid, 0)

        def chunk_body(i, carry):
            c = wid + i * _NW
            for p in (0, 1):

                @pl.when(lax.rem(i, 2) == p)
                def _(p=p):

                    @pl.when((i >= 2) & (c - 2 * _NW < n_full))
                    def _():
                        drain_out(p)

                    @pl.when(c < n_full)
                    def _():
                        drain_in(p)

                        @pl.when(c + _NW < n_full)
                        def _():
                            fire(c + _NW, 1 - p)

                        transpose_block(in_bufs[p], out_bufs[p], _CW)
                        pltpu.async_copy(
                            out_bufs[p],
                            out_hbm.at[pl.ds(c * sup, sup)],
                            osem)

            return carry

        lax.fori_loop(0, n_iter, chunk_body, 0)
        for i in (n_iter - 2, n_iter - 1):

            @pl.when(wid + i * _NW < n_full)
            def _(i=i):
                drain_out(i % 2)

        if tail:

            @pl.when(wid == n_full % _NW)
            def _():
                pltpu.sync_copy(
                    tt_hbm.at[:, pl.ds(n_full * _CW, tail)], int_v)
                transpose_block(int_v, outt_v, tail)
                pltpu.sync_copy(
                    outt_v, out_hbm.at[pl.ds(n_full * sup, tail // 4)])

    return ka


@functools.cache
def _build(B, F, D):
    b_per_w = B // _NW
    g_per_f = b_per_w // _GROUP
    stride = b_per_w + 1
    mesh = plsc.VectorSubcoreMesh(core_axis_name="c", subcore_axis_name="s")

    @functools.partial(
        pl.kernel,
        mesh=mesh,
        out_type=jax.ShapeDtypeStruct((F, D, B), jnp.float32),
        compiler_params=pltpu.CompilerParams(
            use_tc_tiling_on_sc=False, needs_layout_passes=False),
        scratch_types=[
            pltpu.VMEM((32, b_per_w), jnp.int32),
            pltpu.VMEM((32, b_per_w), jnp.float32),
            pltpu.VMEM((b_per_w, D), jnp.float32),
            pltpu.VMEM((b_per_w, D), jnp.float32),
            pltpu.VMEM((D, stride), jnp.float32),
            pltpu.VMEM((D, stride), jnp.float32),
            pltpu.SemaphoreType.DMA,
            pltpu.SemaphoreType.DMA,
            pltpu.SemaphoreType.DMA,
        ],
    )
    def k(idx_hbm, val_hbm, table_hbm, out_hbm,
          idx_v, val_v, rows0_v, rows1_v, outb0_v, outb1_v,
          gsem0, gsem1, osem):
        wid = lax.axis_index("s") * _NC + lax.axis_index("c")
        b0 = wid * b_per_w
        pltpu.sync_copy(idx_hbm.at[:, pl.ds(b0, b_per_w)], idx_v)
        pltpu.sync_copy(val_hbm.at[:, pl.ds(b0, b_per_w)], val_v)
        lane = lax.iota(jnp.int32, 16)
        d_lo = lane
        d_hi = lane + 16
        rows_bufs = [rows0_v, rows1_v]
        out_bufs = [outb0_v, outb1_v]
        g_sems = [gsem0, gsem1]

        def fire(f, p):
            for g in range(g_per_f):
                pltpu.async_copy(
                    table_hbm.at[idx_v.at[f, pl.ds(g * _GROUP, _GROUP)]],
                    rows_bufs[p].at[pl.ds(g * _GROUP, _GROUP)],
                    g_sems[p],
                )

        def drain_gather(p):
            pltpu.make_async_copy(
                table_hbm.at[pl.ds(0, b_per_w)], rows_bufs[p], g_sems[p],
            ).wait()

        def drain_out(p):
            pltpu.make_async_copy(
                out_hbm.at[0, :, pl.ds(0, b_per_w)],
                out_bufs[p].at[:, pl.ds(0, b_per_w)],
                osem,
            ).wait()

        fire(0, 0)

        def field_body(f, carry):
            for p in (0, 1):

                @pl.when(lax.rem(f, 2) == p)
                def _(p=p):
                    drain_gather(p)

                    @pl.when(f + 1 < F)
                    def _():
                        fire(f + 1, 1 - p)

                    @pl.when(f >= 2)
                    def _():
                        drain_out(p)

                    rows_v = rows_bufs[p]
                    outb_v = out_bufs[p]

                    def j_body(j, carry2):
                        vvec = val_v[f, pl.ds(j * 16, 16)]
                        for u in range(16):
                            r = j * 16 + u
                            v = vvec[u]
                            rvec = lane * 0 + r
                            plsc.store_scatter(
                                outb_v, [d_lo, rvec],
                                rows_v[r, pl.ds(0, 16)] * v)
                            plsc.store_scatter(
                                outb_v, [d_hi, rvec],
                                rows_v[r, pl.ds(16, 16)] * v)
                        return carry2

                    lax.fori_loop(0, b_per_w // 16, j_body, 0)
                    pltpu.async_copy(
                        outb_v.at[:, pl.ds(0, b_per_w)],
                        out_hbm.at[f, :, pl.ds(b0, b_per_w)],
                        osem)
            return carry

        lax.fori_loop(0, F, field_body, 0)
        drain_out(0)
        drain_out(1)

    return k


def kernel(cat_index, cat_val, field_size, table):
    B, F = cat_index.shape
    V, D = table.shape
    t_lin = _build_tpose(V, D)(table.T).reshape(V, D)
    idx_t = jnp.pad(cat_index.T.astype(jnp.int32), ((0, 32 - F), (0, 0)))
    val_t = jnp.pad(cat_val.T, ((0, 32 - F), (0, 0)))
    out_t = _build(B, F, D)(idx_t, val_t, t_lin)
    return out_t.transpose(2, 0, 1)


# speedup vs baseline: 1.0738x; 1.0738x over previous
import functools

import jax
import jax.numpy as jnp
from jax import lax
from jax.experimental import pallas as pl
from jax.experimental.pallas import tpu as pltpu
from jax.experimental.pallas import tpu_sc as plsc

_NC = 2
_NS = 16
_NW = _NC * _NS

_GROUP = 128
_TCC = 1024


@functools.cache
def _build_tconv(V, D):
    def body(t_ref, o_ref):
        o_ref[...] = t_ref[...].T.reshape(_TCC * D // 128, 128)

    return pl.pallas_call(
        body,
        grid=(pl.cdiv(V, _TCC),),
        in_specs=[pl.BlockSpec((D, _TCC), lambda g: (0, g))],
        out_specs=pl.BlockSpec((_TCC * D // 128, 128), lambda g: (g, 0)),
        out_shape=jax.ShapeDtypeStruct((V * D // 128, 128), jnp.float32),
    )


@functools.cache
def _build_tpose(V, D):
    _CW = 512
    n_full = V // _CW
    tail = V - n_full * _CW
    n_iter = (n_full + _NW - 1) // _NW
    mesh = plsc.VectorSubcoreMesh(core_axis_name="c", subcore_axis_name="s")

    @functools.partial(
        pl.kernel,
        mesh=mesh,
        out_type=jax.ShapeDtypeStruct((V * D // 128, 128), jnp.float32),
        compiler_params=pltpu.CompilerParams(
            use_tc_tiling_on_sc=True, needs_layout_passes=False),
        scratch_types=[
            pltpu.VMEM((D, _CW), jnp.float32),
            pltpu.VMEM((D, _CW), jnp.float32),
            pltpu.VMEM((_CW * D // 128, 128), jnp.float32),
            pltpu.VMEM((_CW * D // 128, 128), jnp.float32),
            pltpu.VMEM((D, 64), jnp.float32),
            pltpu.VMEM((16, 128), jnp.float32),
            pltpu.SemaphoreType.DMA,
            pltpu.SemaphoreType.DMA,
            pltpu.SemaphoreType.DMA,
        ],
    )
    def ka(tt_hbm, out_hbm, in0_v, in1_v, ob0_v, ob1_v, int_v, outt_v,
           isem0, isem1, osem):
        wid = lax.axis_index("s") * _NC + lax.axis_index("c")
        lane = lax.iota(jnp.int32, 16)
        dvec = [lane, lane + 16]
        in_bufs = [in0_v, in1_v]
        out_bufs = [ob0_v, ob1_v]
        in_sems = [isem0, isem1]
        sup = _CW * D // 128

        mconsts = [(lane + k) & 15 for k in range(16)]

        def transpose_block(src, dst, ncol):
            def cb_body(cb, carry):
                for k in range(16):
                    m = mconsts[k]
                    mrow = lax.shift_right_logical(m, 2)
                    mcol = lax.shift_left(m & 3, 5) + lane
                    lcol = m + cb * 16
                    srow = mrow + cb * 4
                    for d0 in range(D // 16):
                        x = plsc.load_gather(src, [dvec[d0], lcol])
                        plsc.store_scatter(dst, [srow, mcol + d0 * 16], x)
                return carry

            lax.fori_loop(0, ncol // 16, cb_body, 0)

        def fire(c, p):
            pltpu.async_copy(
                tt_hbm.at[:, pl.ds(c * _CW, _CW)], in_bufs[p], in_sems[p])

        def drain_in(p):
            pltpu.make_async_copy(
                tt_hbm.at[:, pl.ds(0, _CW)], in_bufs[p], in_sems[p]).wait()

        def drain_out(p):
            pltpu.make_async_copy(
                out_hbm.at[pl.ds(0, sup)], out_bufs[p], osem).wait()

        @pl.when(wid < n_full)
        def _():
            fire(wid, 0)

        def chunk_body(i, carry):
            c = wid + i * _NW
            for p in (0, 1):

                @pl.when(lax.rem(i, 2) == p)
                def _(p=p):

                    @pl.when((i >= 2) & (c - 2 * _NW < n_full))
                    def _():
                        drain_out(p)

                    @pl.when(c < n_full)
                    def _():
                        drain_in(p)

                        @pl.when(c + _NW < n_full)
                        def _():
                            fire(c + _NW, 1 - p)

                        transpose_block(in_bufs[p], out_bufs[p], _CW)
                        pltpu.async_copy(
                            out_bufs[p],
                            out_hbm.at[pl.ds(c * sup, sup)],
                            osem)

            return carry

        lax.fori_loop(0, n_iter, chunk_body, 0)
        for i in (n_iter - 2, n_iter - 1):

            @pl.when(wid + i * _NW < n_full)
            def _(i=i):
                drain_out(i % 2)

        if tail:

            @pl.when(wid == n_full % _NW)
            def _():
                pltpu.sync_copy(
                    tt_hbm.at[:, pl.ds(n_full * _CW, tail)], int_v)
                transpose_block(int_v, outt_v, tail)
                pltpu.sync_copy(
                    outt_v, out_hbm.at[pl.ds(n_full * sup, tail // 4)])

    return ka


@functools.cache
def _build(B, F, D):
    b_per_w = B // _NW
    g_per_f = b_per_w // _GROUP
    stride = b_per_w + 1
    mesh = plsc.VectorSubcoreMesh(core_axis_name="c", subcore_axis_name="s")

    @functools.partial(
        pl.kernel,
        mesh=mesh,
        out_type=jax.ShapeDtypeStruct((F, D, B), jnp.float32),
        compiler_params=pltpu.CompilerParams(
            use_tc_tiling_on_sc=False, needs_layout_passes=False),
        scratch_types=[
            pltpu.VMEM((32, b_per_w), jnp.int32),
            pltpu.VMEM((32, b_per_w), jnp.float32),
            pltpu.VMEM((b_per_w, D), jnp.float32),
            pltpu.VMEM((b_per_w, D), jnp.float32),
            pltpu.VMEM((D, stride), jnp.float32),
            pltpu.VMEM((D, stride), jnp.float32),
            pltpu.SemaphoreType.DMA,
            pltpu.SemaphoreType.DMA,
            pltpu.SemaphoreType.DMA,
        ],
    )
    def k(idx_hbm, val_hbm, table_hbm, out_hbm,
          idx_v, val_v, rows0_v, rows1_v, outb0_v, outb1_v,
          gsem0, gsem1, osem):
        wid = lax.axis_index("s") * _NC + lax.axis_index("c")
        b0 = wid * b_per_w
        pltpu.sync_copy(idx_hbm.at[:, pl.ds(b0, b_per_w)], idx_v)
        pltpu.sync_copy(val_hbm.at[:, pl.ds(b0, b_per_w)], val_v)
        lane = lax.iota(jnp.int32, 16)
        d_lo = lane
        d_hi = lane + 16
        rows_bufs = [rows0_v, rows1_v]
        out_bufs = [outb0_v, outb1_v]
        g_sems = [gsem0, gsem1]

        def fire(f, p):
            for g in range(g_per_f):
                pltpu.async_copy(
                    table_hbm.at[idx_v.at[f, pl.ds(g * _GROUP, _GROUP)]],
                    rows_bufs[p].at[pl.ds(g * _GROUP, _GROUP)],
                    g_sems[p],
                )

        def drain_gather(p):
            pltpu.make_async_copy(
                table_hbm.at[pl.ds(0, b_per_w)], rows_bufs[p], g_sems[p],
            ).wait()

        def drain_out(p):
            pltpu.make_async_copy(
                out_hbm.at[0, :, pl.ds(0, b_per_w)],
                out_bufs[p].at[:, pl.ds(0, b_per_w)],
                osem,
            ).wait()

        fire(0, 0)

        def field_body(f, carry):
            for p in (0, 1):

                @pl.when(lax.rem(f, 2) == p)
                def _(p=p):
                    drain_gather(p)

                    @pl.when(f + 1 < F)
                    def _():
                        fire(f + 1, 1 - p)

                    @pl.when(f >= 2)
                    def _():
                        drain_out(p)

                    rows_v = rows_bufs[p]
                    outb_v = out_bufs[p]

                    def j_body(j, carry2):
                        vvec = val_v[f, pl.ds(j * 16, 16)]
                        for u in range(16):
                            r = j * 16 + u
                            v = vvec[u]
                            rvec = lane * 0 + r
                            plsc.store_scatter(
                                outb_v, [d_lo, rvec],
                                rows_v[r, pl.ds(0, 16)] * v)
                            plsc.store_scatter(
                                outb_v, [d_hi, rvec],
                                rows_v[r, pl.ds(16, 16)] * v)
                        return carry2

                    lax.fori_loop(0, b_per_w // 16, j_body, 0)
                    pltpu.async_copy(
                        outb_v.at[:, pl.ds(0, b_per_w)],
                        out_hbm.at[f, :, pl.ds(b0, b_per_w)],
                        osem)
            return carry

        lax.fori_loop(0, F, field_body, 0)
        drain_out(0)
        drain_out(1)

    return k


def kernel(cat_index, cat_val, field_size, table):
    B, F = cat_index.shape
    V, D = table.shape
    t_lin = _build_tpose(V, D)(table.T).reshape(V, D)
    idx_t = jnp.pad(cat_index.T.astype(jnp.int32), ((0, 32 - F), (0, 0)))
    val_t = jnp.pad(cat_val.T, ((0, 32 - F), (0, 0)))
    out_t = _build(B, F, D)(idx_t, val_t, t_lin)
    return out_t.transpose(2, 0, 1)
